# baseline (speedup 1.0000x reference)
import functools

import jax
import jax.numpy as jnp
from jax.experimental import pallas as pl
from jax.experimental.pallas import tpu as pltpu


def _body(H, W, a_ref, b_ref, map_a_ref, map_b_ref, s_ref):
    xs = jax.lax.broadcasted_iota(jnp.int32, (1, 1, W), 2).astype(jnp.float32)
    ys = jax.lax.broadcasted_iota(jnp.int32, (1, H, 1), 1).astype(jnp.float32)

    def one(x_ref, map_ref):
        R = x_ref[0]
        mn = jnp.min(R, axis=(1, 2), keepdims=True)
        mx = jnp.max(R, axis=(1, 2), keepdims=True)
        inv = 1.0 / (mx - mn)
        m = (R - mn) * inv
        map_ref[0] = m
        z = jnp.sum(m, axis=(1, 2), keepdims=True)[:, 0, :]
        kx = jnp.sum(m * xs, axis=(1, 2), keepdims=True)[:, 0, :]
        ky = jnp.sum(m * ys, axis=(1, 2), keepdims=True)[:, 0, :]
        return [jnp.round(kx / z), jnp.round(ky / z), z, z]

    cols_a = one(a_ref, map_a_ref)
    cols_b = one(b_ref, map_b_ref)
    s_ref[0] = jnp.concatenate(cols_a + cols_b, axis=1)


@jax.jit
def kernel(combined_hm_preds, tf_combined_hm_preds):
    B, C, H, W = combined_hm_preds.shape
    CB = 32
    nc = C // CB

    big_spec = pl.BlockSpec((1, CB, H, W), lambda b, c: (b, c, 0, 0))
    s_spec = pl.BlockSpec((1, CB, 8), lambda b, c: (b, c, 0))
    big_shape = jax.ShapeDtypeStruct((B, C, H, W), jnp.float32)
    s_shape = jax.ShapeDtypeStruct((B, C, 8), jnp.float32)

    outs = pl.pallas_call(
        functools.partial(_body, H, W),
        grid=(B, nc),
        in_specs=[big_spec, big_spec],
        out_specs=(big_spec, big_spec, s_spec),
        out_shape=(big_shape, big_shape, s_shape),
        compiler_params=pltpu.CompilerParams(
            dimension_semantics=("parallel", "parallel"),
            vmem_limit_bytes=56 * 1024 * 1024,
        ),
        name="minmax_centroid_fused",
    )(combined_hm_preds, tf_combined_hm_preds)

    map_val_all, tf_map_val_all, s = outs
    keypoint = s[:, :, :2]
    get_zeta = s[:, :, 2]
    tf_keypoint = s[:, :, 4:6]
    return (map_val_all, keypoint, get_zeta, tf_map_val_all, tf_keypoint)


# speedup vs baseline: 1.0595x; 1.0595x over previous
import functools

import jax
import jax.numpy as jnp
from jax.experimental import pallas as pl
from jax.experimental.pallas import tpu as pltpu


def _body(H, W, a_ref, b_ref, map_a_ref, map_b_ref, s_ref):
    xs = jax.lax.broadcasted_iota(jnp.int32, (1, 1, W), 2).astype(jnp.float32)
    ys = jax.lax.broadcasted_iota(jnp.int32, (1, H, 1), 1).astype(jnp.float32)

    def one(x_ref, map_ref):
        R = x_ref[0]
        mn = jnp.min(R, axis=(1, 2), keepdims=True)
        mx = jnp.max(R, axis=(1, 2), keepdims=True)
        inv = 1.0 / (mx - mn)
        m = (R - mn) * inv
        map_ref[0] = m
        z = jnp.sum(m, axis=(1, 2), keepdims=True)[:, 0, :]
        kx = jnp.sum(m * xs, axis=(1, 2), keepdims=True)[:, 0, :]
        ky = jnp.sum(m * ys, axis=(1, 2), keepdims=True)[:, 0, :]
        return [jnp.round(kx / z), jnp.round(ky / z), z, z]

    cols_a = one(a_ref, map_a_ref)
    cols_b = one(b_ref, map_b_ref)
    s_ref[0] = jnp.concatenate(cols_a + cols_b, axis=1)


@jax.jit
def kernel(combined_hm_preds, tf_combined_hm_preds):
    B, C, H, W = combined_hm_preds.shape
    CB = 64
    nc = C // CB

    big_spec = pl.BlockSpec((1, CB, H, W), lambda b, c: (b, c, 0, 0))
    s_spec = pl.BlockSpec((1, CB, 8), lambda b, c: (b, c, 0))
    big_shape = jax.ShapeDtypeStruct((B, C, H, W), jnp.float32)
    s_shape = jax.ShapeDtypeStruct((B, C, 8), jnp.float32)

    outs = pl.pallas_call(
        functools.partial(_body, H, W),
        grid=(B, nc),
        in_specs=[big_spec, big_spec],
        out_specs=(big_spec, big_spec, s_spec),
        out_shape=(big_shape, big_shape, s_shape),
        compiler_params=pltpu.CompilerParams(
            dimension_semantics=("parallel", "parallel"),
            vmem_limit_bytes=56 * 1024 * 1024,
        ),
        name="minmax_centroid_fused",
    )(combined_hm_preds, tf_combined_hm_preds)

    map_val_all, tf_map_val_all, s = outs
    keypoint = s[:, :, :2]
    get_zeta = s[:, :, 2]
    tf_keypoint = s[:, :, 4:6]
    return (map_val_all, keypoint, get_zeta, tf_map_val_all, tf_keypoint)
